# baseline (speedup 1.0000x reference)
import jax
import jax.numpy as jnp
from jax.experimental import pallas as pl

_BM = 512


def _masked_matmul_kernel(x_ref, hi_ref, lo_ref, o_ref):
    mask = (x_ref[...] != 0).astype(jnp.bfloat16)
    o_ref[...] = (
        jnp.dot(mask, hi_ref[...], preferred_element_type=jnp.float32)
        + jnp.dot(mask, lo_ref[...], preferred_element_type=jnp.float32)
    )


@jax.jit
def kernel(x, table):
    batch, num_cat = x.shape
    _, embed_dim = table.shape
    hi = table.astype(jnp.bfloat16)
    lo = (table - hi.astype(jnp.float32)).astype(jnp.bfloat16)
    return pl.pallas_call(
        _masked_matmul_kernel,
        grid=(batch // _BM,),
        in_specs=[
            pl.BlockSpec((_BM, num_cat), lambda i: (i, 0)),
            pl.BlockSpec((num_cat, embed_dim), lambda i: (0, 0)),
            pl.BlockSpec((num_cat, embed_dim), lambda i: (0, 0)),
        ],
        out_specs=pl.BlockSpec((_BM, embed_dim), lambda i: (i, 0)),
        out_shape=jax.ShapeDtypeStruct((batch, embed_dim), jnp.float32),
    )(x, hi, lo)


# speedup vs baseline: 1.2197x; 1.2197x over previous
import jax
import jax.numpy as jnp
from jax.experimental import pallas as pl

_STEP = 2048
_SUB = 256
_NSUB = _STEP // _SUB


def _masked_matmul_kernel(*refs):
    x_refs = refs[:_NSUB]
    table_ref = refs[_NSUB]
    o_ref = refs[_NSUB + 1]
    t = table_ref[...]
    for j in range(_NSUB):
        mask = (x_refs[j][...] != 0).astype(jnp.float32)
        o_ref[j * _SUB:(j + 1) * _SUB, :] = jnp.dot(
            mask, t, preferred_element_type=jnp.float32)


@jax.jit
def kernel(x, table):
    batch, num_cat = x.shape
    _, embed_dim = table.shape
    in_specs = [
        pl.BlockSpec((_SUB, num_cat), (lambda i, j=j: (i * _NSUB + j, 0)))
        for j in range(_NSUB)
    ]
    in_specs.append(pl.BlockSpec((num_cat, embed_dim), lambda i: (0, 0)))
    return pl.pallas_call(
        _masked_matmul_kernel,
        grid=(batch // _STEP,),
        in_specs=in_specs,
        out_specs=pl.BlockSpec((_STEP, embed_dim), lambda i: (i, 0)),
        out_shape=jax.ShapeDtypeStruct((batch, embed_dim), jnp.float32),
    )(*([x] * _NSUB), table)


# speedup vs baseline: 3.3932x; 2.7821x over previous
import jax
import jax.numpy as jnp
from jax import lax
from jax.experimental import pallas as pl

_BN = 1024


def _masked_matmul_kernel(xt_ref, table_ref, o_ref):
    mask = (xt_ref[...] != 0).astype(jnp.float32)
    o_ref[...] = lax.dot_general(
        mask, table_ref[...],
        dimension_numbers=(((0,), (0,)), ((), ())),
        preferred_element_type=jnp.float32,
    )


@jax.jit
def kernel(x, table):
    batch, num_cat = x.shape
    _, embed_dim = table.shape
    xt = x.T
    return pl.pallas_call(
        _masked_matmul_kernel,
        grid=(batch // _BN,),
        in_specs=[
            pl.BlockSpec((num_cat, _BN), lambda i: (0, i)),
            pl.BlockSpec((num_cat, embed_dim), lambda i: (0, 0)),
        ],
        out_specs=pl.BlockSpec((_BN, embed_dim), lambda i: (i, 0)),
        out_shape=jax.ShapeDtypeStruct((batch, embed_dim), jnp.float32),
    )(xt, table)


# speedup vs baseline: 3.8832x; 1.1444x over previous
import jax
import jax.numpy as jnp
from jax import lax
from jax.experimental import pallas as pl

_STEP = 2048
_SUB = 512
_NSUB = _STEP // _SUB


def _masked_matmul_kernel(*refs):
    xt_refs = refs[:_NSUB]
    table_ref = refs[_NSUB]
    o_ref = refs[_NSUB + 1]
    t = table_ref[...]
    for j in range(_NSUB):
        mask = (xt_refs[j][...] != 0).astype(jnp.float32)
        o_ref[j * _SUB:(j + 1) * _SUB, :] = lax.dot_general(
            mask, t,
            dimension_numbers=(((0,), (0,)), ((), ())),
            preferred_element_type=jnp.float32,
        )


@jax.jit
def kernel(x, table):
    batch, num_cat = x.shape
    _, embed_dim = table.shape
    xt = x.T
    in_specs = [
        pl.BlockSpec((num_cat, _SUB), (lambda i, j=j: (0, i * _NSUB + j)))
        for j in range(_NSUB)
    ]
    in_specs.append(pl.BlockSpec((num_cat, embed_dim), lambda i: (0, 0)))
    return pl.pallas_call(
        _masked_matmul_kernel,
        grid=(batch // _STEP,),
        in_specs=in_specs,
        out_specs=pl.BlockSpec((_STEP, embed_dim), lambda i: (i, 0)),
        out_shape=jax.ShapeDtypeStruct((batch, embed_dim), jnp.float32),
    )(*([xt] * _NSUB), table)
